# baseline (speedup 1.0000x reference)
import jax
import jax.numpy as jnp
from jax.experimental import pallas as pl

_EPS = 1e-8
_TOPK = 8


def _main_kernel(xn_ref, dn_ref, db_ref, emb_ref, idx_ref):
    s = jax.lax.dot_general(
        xn_ref[...], dn_ref[...],
        (((1,), (1,)), ((), ())), preferred_element_type=jnp.float32)
    n = s.shape[1]
    iota = jax.lax.broadcasted_iota(jnp.int32, s.shape, 1).astype(jnp.float32)
    big = jnp.float32(n)
    s0 = s
    v0 = None
    z = None
    cols = []
    neg = jnp.float32(-jnp.inf)
    for k in range(_TOPK):
        m = jnp.max(s, axis=1, keepdims=True)
        idx = jnp.min(jnp.where(s == m, iota, big), axis=1, keepdims=True)
        if k == 0:
            v0 = m
            z = jnp.ones_like(m)
        else:
            z = z + jnp.exp(m - v0)
        s = jnp.where(iota == idx, neg, s)
        cols.append(idx)
    idx_ref[...] = jnp.concatenate(cols, axis=1).astype(jnp.int32)
    u = jnp.where(s == neg, jnp.exp(s0 - v0), 0.0).astype(jnp.bfloat16)
    g = jax.lax.dot_general(
        u, db_ref[...],
        (((1,), (0,)), ((), ())), preferred_element_type=jnp.float32)
    emb_ref[...] = g / z


def kernel(inputs_flatten, dictionary):
    b, dd = inputs_flatten.shape
    n = dictionary.shape[0]
    blk_b = min(b, 256)

    x_norm = jnp.maximum(
        jnp.linalg.norm(inputs_flatten, axis=1, keepdims=True), _EPS)
    d_norm = jnp.maximum(
        jnp.linalg.norm(dictionary, axis=1, keepdims=True), _EPS)
    xnb = (inputs_flatten / x_norm).astype(jnp.bfloat16)
    dnb = (dictionary / d_norm).astype(jnp.bfloat16)
    db = dictionary.astype(jnp.bfloat16)

    emb, idx = pl.pallas_call(
        _main_kernel,
        grid=(b // blk_b,),
        in_specs=[
            pl.BlockSpec((blk_b, dd), lambda i: (i, 0)),
            pl.BlockSpec((n, dd), lambda i: (0, 0)),
            pl.BlockSpec((n, dd), lambda i: (0, 0)),
        ],
        out_specs=[
            pl.BlockSpec((blk_b, dd), lambda i: (i, 0)),
            pl.BlockSpec((blk_b, _TOPK), lambda i: (i, 0)),
        ],
        out_shape=[
            jax.ShapeDtypeStruct((b, dd), jnp.float32),
            jax.ShapeDtypeStruct((b, _TOPK), jnp.int32),
        ],
    )(xnb, dnb, db)
    return emb, idx
